# baseline (speedup 1.0000x reference)
import math

import numpy as np
import jax
import jax.numpy as jnp
from jax import lax
from jax.experimental import pallas as pl
from jax.experimental.pallas import tpu as pltpu
from jax.experimental.pallas import tpu_sc as plsc

LANES = 16
D_MODEL = 512

_POS_ROW = np.sin(np.arange(D_MODEL, dtype=np.float64) * (math.pi / 180.0)).astype(
    np.float32
)


def _sc_body(xt_hbm, pos_hbm, table_hbm, out_hbm, idx_v, row1_v, rows_v, pos_v, sems):
    wid = lax.axis_index("s") + lax.axis_index("c")

    @pl.when(wid == 0)
    def _():
        tail_cp = pltpu.async_copy(xt_hbm, idx_v, sems.at[0])
        pos_cp = pltpu.async_copy(pos_hbm, pos_v, sems.at[1])
        row1_cp = pltpu.async_copy(table_hbm.at[pl.ds(1, 1)], row1_v, sems.at[2])
        tail_cp.wait()
        gather_cp = pltpu.async_copy(table_hbm.at[idx_v], rows_v, sems.at[3])
        pos_cp.wait()
        row1_cp.wait()
        for c in range(D_MODEL // LANES):
            sl = pl.ds(LANES * c, LANES)
            row1_v[0, sl] += pos_v[sl]
        gather_cp.wait()
        for c in range(D_MODEL // LANES):
            sl = pl.ds(LANES * c, LANES)
            rows_v[LANES - 1, sl] += pos_v[sl]
        out0_cp = pltpu.async_copy(row1_v, out_hbm.at[pl.ds(0, 1)], sems.at[2])
        pltpu.sync_copy(rows_v.at[pl.ds(LANES - 1, 1)], out_hbm.at[pl.ds(1, 1)])
        out0_cp.wait()


def kernel(x, embed_table):
    mesh = plsc.VectorSubcoreMesh(
        core_axis_name="c", subcore_axis_name="s", num_cores=1, num_subcores=1
    )
    return pl.kernel(
        _sc_body,
        out_type=jax.ShapeDtypeStruct((2, D_MODEL), jnp.float32),
        mesh=mesh,
        scratch_types=[
            pltpu.VMEM((LANES,), jnp.int32),
            pltpu.VMEM((1, D_MODEL), jnp.float32),
            pltpu.VMEM((LANES, D_MODEL), jnp.float32),
            pltpu.VMEM((D_MODEL,), jnp.float32),
            pltpu.SemaphoreType.DMA((4,)),
        ],
    )(x[x.shape[0] - LANES :], jnp.asarray(_POS_ROW), embed_table)


# speedup vs baseline: 1.0097x; 1.0097x over previous
import math

import numpy as np
import jax
import jax.numpy as jnp
from jax import lax
from jax.experimental import pallas as pl
from jax.experimental.pallas import tpu as pltpu
from jax.experimental.pallas import tpu_sc as plsc

LANES = 16
D_MODEL = 512

_POS_ROW = np.sin(np.arange(D_MODEL, dtype=np.float64) * (math.pi / 180.0)).astype(
    np.float32
)


def _sc_body(xt_hbm, pos_hbm, table_hbm, out_hbm, idx_v, row1_v, rows_v, pos_v, sems):
    wid = lax.axis_index("s") + lax.axis_index("c")

    @pl.when(wid == 0)
    def _():
        tail_cp = pltpu.async_copy(xt_hbm, idx_v, sems.at[0])
        pos_cp = pltpu.async_copy(pos_hbm, pos_v, sems.at[1])
        row1_cp = pltpu.async_copy(table_hbm.at[pl.ds(1, 1)], row1_v, sems.at[2])
        tail_cp.wait()
        gather_cp = pltpu.async_copy(table_hbm.at[idx_v], rows_v, sems.at[3])
        pos_cp.wait()
        row1_cp.wait()

        def _add_row1(c, carry):
            sl = pl.ds(LANES * c, LANES)
            row1_v[0, sl] += pos_v[sl]
            return carry

        lax.fori_loop(0, D_MODEL // LANES, _add_row1, 0)
        gather_cp.wait()

        def _add_rowx(c, carry):
            sl = pl.ds(LANES * c, LANES)
            rows_v[LANES - 1, sl] += pos_v[sl]
            return carry

        lax.fori_loop(0, D_MODEL // LANES, _add_rowx, 0)
        out0_cp = pltpu.async_copy(row1_v, out_hbm.at[pl.ds(0, 1)], sems.at[2])
        pltpu.sync_copy(rows_v.at[pl.ds(LANES - 1, 1)], out_hbm.at[pl.ds(1, 1)])
        out0_cp.wait()


def kernel(x, embed_table):
    mesh = plsc.VectorSubcoreMesh(
        core_axis_name="c", subcore_axis_name="s", num_cores=1, num_subcores=1
    )
    return pl.kernel(
        _sc_body,
        out_type=jax.ShapeDtypeStruct((2, D_MODEL), jnp.float32),
        mesh=mesh,
        scratch_types=[
            pltpu.VMEM((LANES,), jnp.int32),
            pltpu.VMEM((1, D_MODEL), jnp.float32),
            pltpu.VMEM((LANES, D_MODEL), jnp.float32),
            pltpu.VMEM((D_MODEL,), jnp.float32),
            pltpu.SemaphoreType.DMA((4,)),
        ],
    )(x[x.shape[0] - LANES :], jnp.asarray(_POS_ROW), embed_table)
